# baseline (speedup 1.0000x reference)
import functools

import jax
import jax.numpy as jnp
from jax import lax
from jax.experimental import pallas as pl
from jax.experimental.pallas import tpu as pltpu
from jax.experimental.pallas import tpu_sc as plsc

_EMBED = 32
_SEQ = 128
_POS_ROWS = 64
_LANES = 16
_GRP = 4
_CHUNK = _GRP * _SEQ * _EMBED


def _emb_kernel(patches_hbm, tok_hbm, pos_hbm, out_hbm,
                idx_v, g0, g1, o0, o1, posv, post_v, sem_g, sem_s):
    info = plsc.get_sparse_core_info()
    num_cores = info.num_cores
    num_workers = num_cores * info.num_subcores
    wid = lax.axis_index("s") * num_cores + lax.axis_index("c")

    batch = patches_hbm.shape[0]
    seqs_per_w = batch // num_workers
    n_groups = seqs_per_w // _GRP
    iota = lax.iota(jnp.int32, _LANES)

    pltpu.sync_copy(pos_hbm, posv)

    def post_body(e, _):
        ecol = jnp.full((_LANES,), e, jnp.int32)
        for s0 in range(_SEQ // _LANES):
            rows = jnp.minimum(iota + (s0 * _LANES), _POS_ROWS - 1)
            post_v[e, pl.ds(s0 * _LANES, _LANES)] = plsc.load_gather(
                posv, [rows, ecol])
        return 0

    lax.fori_loop(0, _EMBED, post_body, 0)

    pltpu.sync_copy(patches_hbm.at[pl.ds(wid * seqs_per_w, seqs_per_w)], idx_v)

    gbufs = (g0, g1)
    obufs = (o0, o1)

    def issue_gathers(g, buf):
        for s in range(_GRP):
            pltpu.async_copy(tok_hbm.at[idx_v.at[g * _GRP + s]],
                             buf.at[pl.ds(s * _SEQ, _SEQ)], sem_g)

    def wait_bytes(buf, sem):
        src = (out_hbm.at[pl.ds(0, buf.shape[0])] if len(buf.shape) == 1
               else tok_hbm.at[pl.ds(0, buf.shape[0])])
        pltpu.make_async_copy(src, buf, sem).wait()

    def compute(gc, oc):
        def e_body(e, _):
            ecol = jnp.full((_LANES,), e, jnp.int32)
            ps = [post_v[e, pl.ds(s0 * _LANES, _LANES)]
                  for s0 in range(_SEQ // _LANES)]
            ebase = e * _SEQ
            for s in range(_GRP):
                for s0 in range(_SEQ // _LANES):
                    rows = iota + (s * _SEQ + s0 * _LANES)
                    v = plsc.load_gather(gc, [rows, ecol])
                    oc[pl.ds(ebase + (s * _SEQ * _EMBED + s0 * _LANES),
                             _LANES)] = v + ps[s0]
            return 0

        lax.fori_loop(0, _EMBED, e_body, 0)

    issue_gathers(0, g0)

    def h_body(h, _):
        for b in range(2):
            g = h * 2 + b
            gc, oc = gbufs[b], obufs[b]
            wait_bytes(gc, sem_g)
            @pl.when(h >= 1)
            def _():
                wait_bytes(oc, sem_s)

            @pl.when(g < n_groups - 1)
            def _():
                issue_gathers(g + 1, gbufs[1 - b])
            compute(gc, oc)
            off = (wid * seqs_per_w + g * _GRP) * (_SEQ * _EMBED)
            pltpu.async_copy(oc, out_hbm.at[pl.ds(off, _CHUNK)], sem_s)
        return 0

    lax.fori_loop(0, n_groups // 2, h_body, 0)
    wait_bytes(o0, sem_s)
    wait_bytes(o1, sem_s)


def kernel(patches, token_table, pos_table):
    batch, seq = patches.shape
    vocab, embed = token_table.shape
    idx = patches.astype(jnp.int32)

    mesh = plsc.VectorSubcoreMesh(core_axis_name="c", subcore_axis_name="s")
    n_rows = batch * seq

    run = functools.partial(
        pl.kernel,
        out_type=jax.ShapeDtypeStruct((n_rows * embed,), jnp.float32),
        mesh=mesh,
        scratch_types=[
            pltpu.VMEM((batch // 32, seq), jnp.int32),
            pltpu.VMEM((_GRP * seq, embed), jnp.float32),
            pltpu.VMEM((_GRP * seq, embed), jnp.float32),
            pltpu.VMEM((_CHUNK,), jnp.float32),
            pltpu.VMEM((_CHUNK,), jnp.float32),
            pltpu.VMEM((_POS_ROWS, embed), jnp.float32),
            pltpu.VMEM((embed, seq), jnp.float32),
            pltpu.SemaphoreType.DMA,
            pltpu.SemaphoreType.DMA,
        ],
        compiler_params=pltpu.CompilerParams(use_tc_tiling_on_sc=False,
                                             needs_layout_passes=False),
    )(_emb_kernel)

    out = run(idx, token_table, pos_table)
    return out.reshape(batch, embed, seq).transpose(0, 2, 1)


# speedup vs baseline: 1.5332x; 1.5332x over previous
import functools

import jax
import jax.numpy as jnp
from jax import lax
from jax.experimental import pallas as pl
from jax.experimental.pallas import tpu as pltpu
from jax.experimental.pallas import tpu_sc as plsc

_EMBED = 32
_SEQ = 128
_POS_ROWS = 64
_LANES = 16
_GRP = 4
_CHUNK = _GRP * _SEQ * _EMBED


def _emb_kernel(patches_hbm, tok_hbm, pos_hbm, out_hbm,
                idx_v, g0, g1, o0, o1, posv, post_v, sem_g, sem_s):
    info = plsc.get_sparse_core_info()
    num_cores = info.num_cores
    num_workers = num_cores * info.num_subcores
    wid = lax.axis_index("s") * num_cores + lax.axis_index("c")

    batch = patches_hbm.shape[0]
    seqs_per_w = batch // num_workers
    n_groups = seqs_per_w // _GRP
    iota = lax.iota(jnp.int32, _LANES)

    pltpu.sync_copy(pos_hbm, posv)

    @plsc.parallel_loop(0, _EMBED)
    def post_body(e):
        ecol = jnp.full((_LANES,), e, jnp.int32)
        for s0 in range(_SEQ // _LANES):
            rows = jnp.minimum(iota + (s0 * _LANES), _POS_ROWS - 1)
            post_v[e, pl.ds(s0 * _LANES, _LANES)] = plsc.load_gather(
                posv, [rows, ecol])

    pltpu.sync_copy(patches_hbm.at[pl.ds(wid * seqs_per_w, seqs_per_w)], idx_v)

    gbufs = (g0, g1)
    obufs = (o0, o1)

    def issue_gathers(g, buf):
        for s in range(_GRP):
            pltpu.async_copy(tok_hbm.at[idx_v.at[g * _GRP + s]],
                             buf.at[pl.ds(s * _SEQ, _SEQ)], sem_g)

    def wait_bytes(buf, sem):
        src = (out_hbm.at[pl.ds(0, buf.shape[0])] if len(buf.shape) == 1
               else tok_hbm.at[pl.ds(0, buf.shape[0])])
        pltpu.make_async_copy(src, buf, sem).wait()

    def compute(gc, oc):
        @plsc.parallel_loop(0, _EMBED, unroll=2)
        def e_body(e):
            ecol = jnp.full((_LANES,), e, jnp.int32)
            ps = [post_v[e, pl.ds(s0 * _LANES, _LANES)]
                  for s0 in range(_SEQ // _LANES)]
            ebase = e * _SEQ
            for s in range(_GRP):
                for s0 in range(_SEQ // _LANES):
                    rows = iota + (s * _SEQ + s0 * _LANES)
                    v = plsc.load_gather(gc, [rows, ecol])
                    oc[pl.ds(ebase + (s * _SEQ * _EMBED + s0 * _LANES),
                             _LANES)] = v + ps[s0]

    issue_gathers(0, g0)

    def h_body(h, _):
        for b in range(2):
            g = h * 2 + b
            gc, oc = gbufs[b], obufs[b]
            wait_bytes(gc, sem_g)
            @pl.when(h >= 1)
            def _():
                wait_bytes(oc, sem_s)

            @pl.when(g < n_groups - 1)
            def _():
                issue_gathers(g + 1, gbufs[1 - b])
            compute(gc, oc)
            off = (wid * seqs_per_w + g * _GRP) * (_SEQ * _EMBED)
            pltpu.async_copy(oc, out_hbm.at[pl.ds(off, _CHUNK)], sem_s)
        return 0

    lax.fori_loop(0, n_groups // 2, h_body, 0)
    wait_bytes(o0, sem_s)
    wait_bytes(o1, sem_s)


def kernel(patches, token_table, pos_table):
    batch, seq = patches.shape
    vocab, embed = token_table.shape
    idx = patches.astype(jnp.int32)

    mesh = plsc.VectorSubcoreMesh(core_axis_name="c", subcore_axis_name="s")
    n_rows = batch * seq

    run = functools.partial(
        pl.kernel,
        out_type=jax.ShapeDtypeStruct((n_rows * embed,), jnp.float32),
        mesh=mesh,
        scratch_types=[
            pltpu.VMEM((batch // 32, seq), jnp.int32),
            pltpu.VMEM((_GRP * seq, embed), jnp.float32),
            pltpu.VMEM((_GRP * seq, embed), jnp.float32),
            pltpu.VMEM((_CHUNK,), jnp.float32),
            pltpu.VMEM((_CHUNK,), jnp.float32),
            pltpu.VMEM((_POS_ROWS, embed), jnp.float32),
            pltpu.VMEM((embed, seq), jnp.float32),
            pltpu.SemaphoreType.DMA,
            pltpu.SemaphoreType.DMA,
        ],
        compiler_params=pltpu.CompilerParams(use_tc_tiling_on_sc=False,
                                             needs_layout_passes=False),
    )(_emb_kernel)

    out = run(idx, token_table, pos_table)
    return out.reshape(batch, embed, seq).transpose(0, 2, 1)


# speedup vs baseline: 1.7813x; 1.1618x over previous
import functools

import jax
import jax.numpy as jnp
from jax import lax
from jax.experimental import pallas as pl
from jax.experimental.pallas import tpu as pltpu
from jax.experimental.pallas import tpu_sc as plsc

_EMBED = 32
_SEQ = 128
_POS_ROWS = 64
_LANES = 16
_GRP = 4
_CHUNK = _GRP * _SEQ * _EMBED


def _emb_kernel(patches_hbm, tok_hbm, pos_hbm, out_hbm,
                idx_v, g0, g1, o0, o1, posv, post_v, sem_g, sem_s):
    info = plsc.get_sparse_core_info()
    num_cores = info.num_cores
    num_workers = num_cores * info.num_subcores
    wid = lax.axis_index("s") * num_cores + lax.axis_index("c")

    batch = patches_hbm.shape[0]
    seqs_per_w = batch // num_workers
    n_groups = seqs_per_w // _GRP
    iota = lax.iota(jnp.int32, _LANES)

    pltpu.sync_copy(pos_hbm, posv)

    @plsc.parallel_loop(0, _EMBED)
    def post_body(e):
        ecol = jnp.full((_LANES,), e, jnp.int32)
        for s0 in range(_SEQ // _LANES):
            rows = jnp.minimum(iota + (s0 * _LANES), _POS_ROWS - 1)
            post_v[e, pl.ds(s0 * _LANES, _LANES)] = plsc.load_gather(
                posv, [rows, ecol])

    pltpu.sync_copy(patches_hbm.at[pl.ds(wid * seqs_per_w, seqs_per_w)], idx_v)

    gbufs = (g0, g1)
    obufs = (o0, o1)

    def issue_gathers(g, buf):
        for s in range(_GRP):
            pltpu.async_copy(tok_hbm.at[idx_v.at[g * _GRP + s]],
                             buf.at[pl.ds(s * _SEQ, _SEQ)], sem_g)

    def wait_bytes(buf, sem):
        src = (out_hbm.at[pl.ds(0, buf.shape[0])] if len(buf.shape) == 1
               else tok_hbm.at[pl.ds(0, buf.shape[0])])
        pltpu.make_async_copy(src, buf, sem).wait()

    def compute(gc, oc):
        @plsc.parallel_loop(0, _EMBED, unroll=2)
        def e_body(e):
            ps = [post_v[e, pl.ds(s0 * _LANES, _LANES)]
                  for s0 in range(_SEQ // _LANES)]
            ebase = e * _SEQ
            for s in range(_GRP):
                for s0 in range(_SEQ // _LANES):
                    col = (s * _SEQ + s0 * _LANES) * _EMBED + e
                    v = plsc.load_gather(gc, [iota, jnp.full((_LANES,), col,
                                                             jnp.int32)])
                    oc[pl.ds(ebase + (s * _SEQ * _EMBED + s0 * _LANES),
                             _LANES)] = v + ps[s0]

    issue_gathers(0, g0)

    def h_body(h, _):
        for b in range(2):
            g = h * 2 + b
            gc, oc = gbufs[b], obufs[b]
            wait_bytes(gc, sem_g)
            @pl.when(h >= 1)
            def _():
                wait_bytes(oc, sem_s)

            @pl.when(g < n_groups - 1)
            def _():
                issue_gathers(g + 1, gbufs[1 - b])
            compute(gc, oc)
            off = (wid * seqs_per_w + g * _GRP) * (_SEQ * _EMBED)
            pltpu.async_copy(oc, out_hbm.at[pl.ds(off, _CHUNK)], sem_s)
        return 0

    lax.fori_loop(0, n_groups // 2, h_body, 0)
    wait_bytes(o0, sem_s)
    wait_bytes(o1, sem_s)


def kernel(patches, token_table, pos_table):
    batch, seq = patches.shape
    vocab, embed = token_table.shape
    idx = patches.astype(jnp.int32)

    mesh = plsc.VectorSubcoreMesh(core_axis_name="c", subcore_axis_name="s")
    n_rows = batch * seq

    run = functools.partial(
        pl.kernel,
        out_type=jax.ShapeDtypeStruct((n_rows * embed,), jnp.float32),
        mesh=mesh,
        scratch_types=[
            pltpu.VMEM((batch // 32, seq), jnp.int32),
            pltpu.VMEM((_GRP * seq, embed), jnp.float32),
            pltpu.VMEM((_GRP * seq, embed), jnp.float32),
            pltpu.VMEM((_CHUNK,), jnp.float32),
            pltpu.VMEM((_CHUNK,), jnp.float32),
            pltpu.VMEM((_POS_ROWS, embed), jnp.float32),
            pltpu.VMEM((embed, seq), jnp.float32),
            pltpu.SemaphoreType.DMA,
            pltpu.SemaphoreType.DMA,
        ],
        compiler_params=pltpu.CompilerParams(use_tc_tiling_on_sc=False,
                                             needs_layout_passes=False),
    )(_emb_kernel)

    out = run(idx, token_table, pos_table)
    return out.reshape(batch, embed, seq).transpose(0, 2, 1)


# speedup vs baseline: 1.8851x; 1.0583x over previous
import functools

import jax
import jax.numpy as jnp
from jax import lax
from jax.experimental import pallas as pl
from jax.experimental.pallas import tpu as pltpu
from jax.experimental.pallas import tpu_sc as plsc

_EMBED = 32
_SEQ = 128
_POS_ROWS = 64
_LANES = 16
_GRP = 4
_CHUNK = _GRP * _SEQ * _EMBED


def _emb_kernel(patches_hbm, tok_hbm, pos_hbm, out_hbm,
                idx_v, g0, g1, o0, o1, posv, post_v, sem_g, sem_s):
    info = plsc.get_sparse_core_info()
    num_cores = info.num_cores
    num_workers = num_cores * info.num_subcores
    wid = lax.axis_index("s") * num_cores + lax.axis_index("c")

    batch = patches_hbm.shape[0]
    seqs_per_w = batch // num_workers
    n_groups = seqs_per_w // _GRP
    iota = lax.iota(jnp.int32, _LANES)

    pltpu.sync_copy(pos_hbm, posv)

    @plsc.parallel_loop(0, _EMBED)
    def post_body(e):
        ecol = jnp.full((_LANES,), e, jnp.int32)
        for s0 in range(_SEQ // _LANES):
            rows = jnp.minimum(iota + (s0 * _LANES), _POS_ROWS - 1)
            post_v[e, pl.ds(s0 * _LANES, _LANES)] = plsc.load_gather(
                posv, [rows, ecol])

    pltpu.sync_copy(patches_hbm.at[pl.ds(wid * seqs_per_w, seqs_per_w)], idx_v)

    gbufs = (g0, g1)
    obufs = (o0, o1)

    def issue_gathers(g, buf):
        for s in range(_GRP):
            pltpu.async_copy(tok_hbm.at[idx_v.at[g * _GRP + s]],
                             buf.at[pl.ds(s * _SEQ, _SEQ)], sem_g)

    def wait_bytes(buf, sem):
        src = (out_hbm.at[pl.ds(0, buf.shape[0])] if len(buf.shape) == 1
               else tok_hbm.at[pl.ds(0, buf.shape[0])])
        pltpu.make_async_copy(src, buf, sem).wait()

    def compute(gc, oc):
        @plsc.parallel_loop(0, _EMBED, unroll=2)
        def e_body(e):
            ecol = jnp.full((_LANES,), e, jnp.int32)
            ebase = e * _SEQ
            for s0 in range(_SEQ // _LANES):
                p = post_v[e, pl.ds(s0 * _LANES, _LANES)]
                for s in range(_GRP):
                    base = s * _SEQ + s0 * _LANES
                    v = plsc.load_gather(gc.at[pl.ds(base, _LANES)],
                                         [iota, ecol])
                    oc[pl.ds(ebase + (s * _SEQ * _EMBED + s0 * _LANES),
                             _LANES)] = v + p

    issue_gathers(0, g0)

    def h_body(h, _):
        for b in range(2):
            g = h * 2 + b
            gc, oc = gbufs[b], obufs[b]
            wait_bytes(gc, sem_g)
            @pl.when(h >= 1)
            def _():
                wait_bytes(oc, sem_s)

            @pl.when(g < n_groups - 1)
            def _():
                issue_gathers(g + 1, gbufs[1 - b])
            compute(gc, oc)
            off = (wid * seqs_per_w + g * _GRP) * (_SEQ * _EMBED)
            pltpu.async_copy(oc, out_hbm.at[pl.ds(off, _CHUNK)], sem_s)
        return 0

    lax.fori_loop(0, n_groups // 2, h_body, 0)
    wait_bytes(o0, sem_s)
    wait_bytes(o1, sem_s)


def kernel(patches, token_table, pos_table):
    batch, seq = patches.shape
    vocab, embed = token_table.shape
    idx = patches.astype(jnp.int32)

    mesh = plsc.VectorSubcoreMesh(core_axis_name="c", subcore_axis_name="s")
    n_rows = batch * seq

    run = functools.partial(
        pl.kernel,
        out_type=jax.ShapeDtypeStruct((n_rows * embed,), jnp.float32),
        mesh=mesh,
        scratch_types=[
            pltpu.VMEM((batch // 32, seq), jnp.int32),
            pltpu.VMEM((_GRP * seq, embed), jnp.float32),
            pltpu.VMEM((_GRP * seq, embed), jnp.float32),
            pltpu.VMEM((_CHUNK,), jnp.float32),
            pltpu.VMEM((_CHUNK,), jnp.float32),
            pltpu.VMEM((_POS_ROWS, embed), jnp.float32),
            pltpu.VMEM((embed, seq), jnp.float32),
            pltpu.SemaphoreType.DMA,
            pltpu.SemaphoreType.DMA,
        ],
        compiler_params=pltpu.CompilerParams(use_tc_tiling_on_sc=False,
                                             needs_layout_passes=False),
    )(_emb_kernel)

    out = run(idx, token_table, pos_table)
    return out.reshape(batch, embed, seq).transpose(0, 2, 1)
